# baseline (speedup 1.0000x reference)
import jax
import jax.numpy as jnp
from jax import lax
from jax.experimental import pallas as pl
from jax.experimental.pallas import tpu as pltpu
from jax.experimental.pallas import tpu_sc as plsc

C = 32
M = 256
N_GEN = 16
N_GRAPHS = 512
NS = 16
LANES = 16
TABLE_ROWS = 512
CHUNK = 6400
HALF = CHUNK // 2


def _table_body(b_ref, pi_ref, out_ref):
    bt = b_ref[...]
    bmax = jnp.max(bt, axis=1, keepdims=True)
    be = jnp.exp(bt - bmax)
    sm_b = be / jnp.sum(be, axis=1, keepdims=True)
    pi = pi_ref[...]
    pmax = jnp.max(pi, axis=0, keepdims=True)
    pe = jnp.exp(pi - pmax)
    sm_pi = pe / jnp.sum(pe, axis=0, keepdims=True)
    num = sm_pi[:, None, :] * sm_b
    denom = jnp.sum(num, axis=0)
    plog = jnp.sum(num * jnp.log(num), axis=0)
    likt = plog / denom
    out_ref[:M, :] = -likt
    out_ref[M:, :] = jnp.zeros((TABLE_ROWS - M, N_GEN), jnp.float32)


def _make_table(b, pi):
    return pl.pallas_call(
        _table_body,
        out_shape=jax.ShapeDtypeStruct((TABLE_ROWS, N_GEN), jnp.float32),
    )(b, pi)


def _sc_body(table_hbm, x_hbm, b_hbm, out_hbm,
             table_v, x_v, b_v, acc_v, buf_v, sum_v, shared, sem):
    sid = lax.axis_index("s")
    base = sid * CHUNK

    cps = [
        pltpu.async_copy(table_hbm, table_v, sem),
        pltpu.async_copy(x_hbm.at[pl.ds(base, CHUNK)], x_v, sem),
        pltpu.async_copy(b_hbm.at[pl.ds(base, CHUNK)], b_v, sem),
    ]

    zeros16 = jnp.zeros((LANES,), jnp.float32)

    @plsc.parallel_loop(0, N_GRAPHS, unroll=4)
    def _(i):
        acc_v[pl.ds(i * LANES, LANES)] = zeros16

    for cp in cps:
        cp.wait()

    iota = lax.iota(jnp.int32, LANES)
    dnums = lax.GatherDimensionNumbers(
        offset_dims=(), collapsed_slice_dims=(0,), start_index_map=(0,))

    def _splat(vec, nn):
        return lax.gather(
            vec, jnp.full((LANES, 1), nn, jnp.int32), dnums, (1,),
            mode=lax.GatherScatterMode.PROMISE_IN_BOUNDS)

    @plsc.parallel_loop(0, HALF // LANES)
    def _(g):
        off = g * LANES
        xva = x_v[pl.ds(off, LANES)]
        bva = b_v[pl.ds(off, LANES)] * N_GEN
        xvb = x_v[pl.ds(HALF + off, LANES)]
        bvb = b_v[pl.ds(HALF + off, LANES)] * N_GEN
        for nn in range(LANES):
            row_a = plsc.load_gather(table_v, [_splat(xva, nn), iota])
            plsc.addupdate_scatter(acc_v, [_splat(bva, nn) + iota], row_a)
            row_b = plsc.load_gather(table_v, [_splat(xvb, nn), iota])
            plsc.addupdate_scatter(acc_v, [_splat(bvb, nn) + iota], row_b)

    plsc.subcore_barrier()
    pltpu.sync_copy(acc_v, shared.at[sid])
    plsc.subcore_barrier()

    span = N_GRAPHS * N_GEN // NS
    for s in range(NS):
        pltpu.sync_copy(shared.at[s, pl.ds(sid * span, span)], buf_v)
        for r in range(span // LANES):
            sl = pl.ds(r * LANES, LANES)
            if s == 0:
                sum_v[sl] = buf_v[sl]
            else:
                sum_v[sl] = sum_v[sl] + buf_v[sl]
    pltpu.sync_copy(sum_v, out_hbm.at[pl.ds(sid * span, span)])


def _make_sc():
    mesh = plsc.VectorSubcoreMesh(
        core_axis_name="c", subcore_axis_name="s", num_cores=1)
    span = N_GRAPHS * N_GEN // NS
    return pl.kernel(
        _sc_body,
        out_type=jax.ShapeDtypeStruct((N_GRAPHS * N_GEN,), jnp.float32),
        mesh=mesh,
        scratch_types=[
            pltpu.VMEM((TABLE_ROWS, N_GEN), jnp.float32),
            pltpu.VMEM((CHUNK,), jnp.int32),
            pltpu.VMEM((CHUNK,), jnp.int32),
            pltpu.VMEM((N_GRAPHS * N_GEN,), jnp.float32),
            pltpu.VMEM((span,), jnp.float32),
            pltpu.VMEM((span,), jnp.float32),
            pltpu.VMEM_SHARED((NS, N_GRAPHS * N_GEN), jnp.float32),
            pltpu.SemaphoreType.DMA,
        ],
        compiler_params=pltpu.CompilerParams(
            use_tc_tiling_on_sc=False, needs_layout_passes=False),
    )


@jax.jit
def kernel(x, edge_index, batch, B, Pi):
    del edge_index
    n = x.shape[0]
    pad = NS * CHUNK - n
    x_pad = jnp.pad(x, (0, pad), constant_values=M)
    b_pad = jnp.pad(batch, (0, pad))

    table = _make_table(B, Pi)
    out = _make_sc()(table, x_pad, b_pad)
    return out.reshape(N_GRAPHS, 1, N_GEN)


# speedup vs baseline: 1.0121x; 1.0121x over previous
import jax
import jax.numpy as jnp
from jax import lax
from jax.experimental import pallas as pl
from jax.experimental.pallas import tpu as pltpu
from jax.experimental.pallas import tpu_sc as plsc

C = 32
M = 256
N_GEN = 16
N_GRAPHS = 512
NS = 16
LANES = 16
TABLE_ROWS = 256
N_NODES_TOTAL = 100000
CHUNK = 6272
TAIL = N_NODES_TOTAL - 15 * CHUNK
HALF = CHUNK // 2
TAIL_HALF = TAIL // 2


def _table_body(b_ref, pi_ref, out_ref):
    bt = b_ref[...]
    bmax = jnp.max(bt, axis=1, keepdims=True)
    be = jnp.exp(bt - bmax)
    sm_b = be / jnp.sum(be, axis=1, keepdims=True)
    pi = pi_ref[...]
    pmax = jnp.max(pi, axis=0, keepdims=True)
    pe = jnp.exp(pi - pmax)
    sm_pi = pe / jnp.sum(pe, axis=0, keepdims=True)
    num = sm_pi[:, None, :] * sm_b
    denom = jnp.sum(num, axis=0)
    plog = jnp.sum(num * jnp.log(num), axis=0)
    likt = plog / denom
    out_ref[...] = -likt


def _make_table(b, pi):
    return pl.pallas_call(
        _table_body,
        out_shape=jax.ShapeDtypeStruct((TABLE_ROWS, N_GEN), jnp.float32),
    )(b, pi)


def _sc_body(table_hbm, x_hbm, b_hbm, out_hbm,
             table_v, x_v, b_v, acc_v, buf_v, sum_v, shared, sem):
    sid = lax.axis_index("s")
    base = sid * CHUNK

    table_cp = pltpu.async_copy(table_hbm, table_v, sem)

    @pl.when(sid < NS - 1)
    def _():
        pltpu.sync_copy(x_hbm.at[pl.ds(base, CHUNK)], x_v)
        pltpu.sync_copy(b_hbm.at[pl.ds(base, CHUNK)], b_v)

    @pl.when(sid == NS - 1)
    def _():
        pltpu.sync_copy(x_hbm.at[pl.ds(base, TAIL)], x_v.at[pl.ds(0, TAIL)])
        pltpu.sync_copy(b_hbm.at[pl.ds(base, TAIL)], b_v.at[pl.ds(0, TAIL)])

    zeros16 = jnp.zeros((LANES,), jnp.float32)

    @plsc.parallel_loop(0, N_GRAPHS, unroll=4)
    def _(i):
        acc_v[pl.ds(i * LANES, LANES)] = zeros16

    table_cp.wait()

    iota = lax.iota(jnp.int32, LANES)
    dnums = lax.GatherDimensionNumbers(
        offset_dims=(), collapsed_slice_dims=(0,), start_index_map=(0,))

    def _splat(vec, nn):
        return lax.gather(
            vec, jnp.full((LANES, 1), nn, jnp.int32), dnums, (1,),
            mode=lax.GatherScatterMode.PROMISE_IN_BOUNDS)

    is_tail = sid == NS - 1
    hoff = jnp.where(is_tail, TAIL_HALF, HALF)
    ngroups = jnp.where(is_tail, TAIL_HALF // LANES, HALF // LANES)

    @plsc.parallel_loop(0, ngroups)
    def _(g):
        off = g * LANES
        xva = x_v[pl.ds(off, LANES)]
        bva = b_v[pl.ds(off, LANES)] * N_GEN
        xvb = x_v[pl.ds(hoff + off, LANES)]
        bvb = b_v[pl.ds(hoff + off, LANES)] * N_GEN
        for nn in range(LANES):
            row_a = plsc.load_gather(table_v, [_splat(xva, nn), iota])
            plsc.addupdate_scatter(acc_v, [_splat(bva, nn) + iota], row_a)
            row_b = plsc.load_gather(table_v, [_splat(xvb, nn), iota])
            plsc.addupdate_scatter(acc_v, [_splat(bvb, nn) + iota], row_b)

    plsc.subcore_barrier()
    pltpu.sync_copy(acc_v, shared.at[sid])
    plsc.subcore_barrier()

    span = N_GRAPHS * N_GEN // NS
    for s in range(NS):
        pltpu.sync_copy(shared.at[s, pl.ds(sid * span, span)], buf_v)
        for r in range(span // LANES):
            sl = pl.ds(r * LANES, LANES)
            if s == 0:
                sum_v[sl] = buf_v[sl]
            else:
                sum_v[sl] = sum_v[sl] + buf_v[sl]
    pltpu.sync_copy(sum_v, out_hbm.at[pl.ds(sid * span, span)])


def _make_sc():
    mesh = plsc.VectorSubcoreMesh(
        core_axis_name="c", subcore_axis_name="s", num_cores=1)
    span = N_GRAPHS * N_GEN // NS
    return pl.kernel(
        _sc_body,
        out_type=jax.ShapeDtypeStruct((N_GRAPHS * N_GEN,), jnp.float32),
        mesh=mesh,
        scratch_types=[
            pltpu.VMEM((TABLE_ROWS, N_GEN), jnp.float32),
            pltpu.VMEM((CHUNK,), jnp.int32),
            pltpu.VMEM((CHUNK,), jnp.int32),
            pltpu.VMEM((N_GRAPHS * N_GEN,), jnp.float32),
            pltpu.VMEM((span,), jnp.float32),
            pltpu.VMEM((span,), jnp.float32),
            pltpu.VMEM_SHARED((NS, N_GRAPHS * N_GEN), jnp.float32),
            pltpu.SemaphoreType.DMA,
        ],
        compiler_params=pltpu.CompilerParams(
            use_tc_tiling_on_sc=False, needs_layout_passes=False),
    )


@jax.jit
def kernel(x, edge_index, batch, B, Pi):
    del edge_index
    table = _make_table(B, Pi)
    out = _make_sc()(table, x, batch)
    return out.reshape(N_GRAPHS, 1, N_GEN)


# speedup vs baseline: 1.0241x; 1.0118x over previous
import jax
import jax.numpy as jnp
from jax import lax
from jax.experimental import pallas as pl
from jax.experimental.pallas import tpu as pltpu
from jax.experimental.pallas import tpu_sc as plsc

C = 32
M = 256
N_GEN = 16
N_GRAPHS = 512
NS = 16
LANES = 16
TABLE_ROWS = 256
N_NODES_TOTAL = 100000
CHUNK = 6272
TAIL = N_NODES_TOTAL - 15 * CHUNK
HALF = CHUNK // 2
TAIL_HALF = TAIL // 2


def _table_body(b_ref, pi_ref, out_ref):
    bt = b_ref[...]
    bmax = jnp.max(bt, axis=1, keepdims=True)
    be = jnp.exp(bt - bmax)
    sm_b = be / jnp.sum(be, axis=1, keepdims=True)
    pi = pi_ref[...]
    pmax = jnp.max(pi, axis=0, keepdims=True)
    pe = jnp.exp(pi - pmax)
    sm_pi = pe / jnp.sum(pe, axis=0, keepdims=True)
    num = sm_pi[:, None, :] * sm_b
    denom = jnp.sum(num, axis=0)
    plog = jnp.sum(num * jnp.log(num), axis=0)
    likt = plog / denom
    out_ref[...] = -likt


def _make_table(b, pi):
    return pl.pallas_call(
        _table_body,
        out_shape=jax.ShapeDtypeStruct((TABLE_ROWS, N_GEN), jnp.float32),
    )(b, pi)


def _sc_body(table_hbm, x_hbm, b_hbm, out_hbm,
             table_v, x_v, b_v, acc_v, buf_v, sum_v, shared, sem):
    sid = lax.axis_index("s")
    base = sid * CHUNK

    table_cp = pltpu.async_copy(table_hbm, table_v, sem)

    @pl.when(sid < NS - 1)
    def _():
        pltpu.sync_copy(x_hbm.at[pl.ds(base, CHUNK)], x_v)
        pltpu.sync_copy(b_hbm.at[pl.ds(base, CHUNK)], b_v)

    @pl.when(sid == NS - 1)
    def _():
        pltpu.sync_copy(x_hbm.at[pl.ds(base, TAIL)], x_v.at[pl.ds(0, TAIL)])
        pltpu.sync_copy(b_hbm.at[pl.ds(base, TAIL)], b_v.at[pl.ds(0, TAIL)])

    zeros16 = jnp.zeros((LANES,), jnp.float32)

    @plsc.parallel_loop(0, N_GRAPHS, unroll=4)
    def _(i):
        acc_v[pl.ds(i * LANES, LANES)] = zeros16

    table_cp.wait()

    iota = lax.iota(jnp.int32, LANES)
    dnums = lax.GatherDimensionNumbers(
        offset_dims=(), collapsed_slice_dims=(0,), start_index_map=(0,))

    def _splat(vec, nn):
        return lax.gather(
            vec, jnp.full((LANES, 1), nn, jnp.int32), dnums, (1,),
            mode=lax.GatherScatterMode.PROMISE_IN_BOUNDS)

    is_tail = sid == NS - 1
    hoff = jnp.where(is_tail, TAIL_HALF, HALF)
    ngroups = jnp.where(is_tail, TAIL_HALF // LANES, HALF // LANES)

    @plsc.parallel_loop(0, ngroups)
    def _(g):
        off = g * LANES
        xva = x_v[pl.ds(off, LANES)]
        bva = b_v[pl.ds(off, LANES)] * N_GEN
        xvb = x_v[pl.ds(hoff + off, LANES)]
        bvb = b_v[pl.ds(hoff + off, LANES)] * N_GEN
        for half in range(2):
            rows = []
            tgts = []
            for nn in range(half * (LANES // 2), (half + 1) * (LANES // 2)):
                rows.append(plsc.load_gather(
                    table_v, [_splat(xva, nn), iota]))
                tgts.append(_splat(bva, nn) + iota)
                rows.append(plsc.load_gather(
                    table_v, [_splat(xvb, nn), iota]))
                tgts.append(_splat(bvb, nn) + iota)
            for row, tgt in zip(rows, tgts):
                plsc.addupdate_scatter(acc_v, [tgt], row)

    plsc.subcore_barrier()
    pltpu.sync_copy(acc_v, shared.at[sid])
    plsc.subcore_barrier()

    span = N_GRAPHS * N_GEN // NS
    for s in range(NS):
        pltpu.sync_copy(shared.at[s, pl.ds(sid * span, span)], buf_v)
        for r in range(span // LANES):
            sl = pl.ds(r * LANES, LANES)
            if s == 0:
                sum_v[sl] = buf_v[sl]
            else:
                sum_v[sl] = sum_v[sl] + buf_v[sl]
    pltpu.sync_copy(sum_v, out_hbm.at[pl.ds(sid * span, span)])


def _make_sc():
    mesh = plsc.VectorSubcoreMesh(
        core_axis_name="c", subcore_axis_name="s", num_cores=1)
    span = N_GRAPHS * N_GEN // NS
    return pl.kernel(
        _sc_body,
        out_type=jax.ShapeDtypeStruct((N_GRAPHS * N_GEN,), jnp.float32),
        mesh=mesh,
        scratch_types=[
            pltpu.VMEM((TABLE_ROWS, N_GEN), jnp.float32),
            pltpu.VMEM((CHUNK,), jnp.int32),
            pltpu.VMEM((CHUNK,), jnp.int32),
            pltpu.VMEM((N_GRAPHS * N_GEN,), jnp.float32),
            pltpu.VMEM((span,), jnp.float32),
            pltpu.VMEM((span,), jnp.float32),
            pltpu.VMEM_SHARED((NS, N_GRAPHS * N_GEN), jnp.float32),
            pltpu.SemaphoreType.DMA,
        ],
        compiler_params=pltpu.CompilerParams(
            use_tc_tiling_on_sc=False, needs_layout_passes=False),
    )


@jax.jit
def kernel(x, edge_index, batch, B, Pi):
    del edge_index
    table = _make_table(B, Pi)
    out = _make_sc()(table, x, batch)
    return out.reshape(N_GRAPHS, 1, N_GEN)


# speedup vs baseline: 1.0505x; 1.0258x over previous
import jax
import jax.numpy as jnp
from jax import lax
from jax.experimental import pallas as pl
from jax.experimental.pallas import tpu as pltpu
from jax.experimental.pallas import tpu_sc as plsc

C = 32
M = 256
N_GEN = 16
N_GRAPHS = 512
NS = 16
LANES = 16
TABLE_ROWS = 512
CHUNK = 6400


def _table_body(bt_ref, pi_ref, out_ref):
    bt = bt_ref[...]
    bmax = jnp.max(bt, axis=2, keepdims=True)
    be = jnp.exp(bt - bmax)
    sm_b = be / jnp.sum(be, axis=2, keepdims=True)
    pi = pi_ref[...]
    pmax = jnp.max(pi, axis=0, keepdims=True)
    pe = jnp.exp(pi - pmax)
    sm_pi = pe / jnp.sum(pe, axis=0, keepdims=True)
    num = sm_pi[:, :, None] * sm_b
    denom = jnp.sum(num, axis=0)
    plog = jnp.sum(num * jnp.log(num), axis=0)
    likt = plog / denom
    out_ref[:M, :] = -likt.T
    out_ref[M:, :] = jnp.zeros((TABLE_ROWS - M, N_GEN), jnp.float32)


def _make_table(b_t, pi):
    return pl.pallas_call(
        _table_body,
        out_shape=jax.ShapeDtypeStruct((TABLE_ROWS, N_GEN), jnp.float32),
    )(b_t, pi)


def _sc_body(table_hbm, x_hbm, b_hbm, out_hbm,
             table_v, x_v, b_v, acc_v, buf_v, sum_v, shared):
    sid = lax.axis_index("s")
    base = sid * CHUNK

    pltpu.sync_copy(table_hbm, table_v)
    pltpu.sync_copy(x_hbm.at[pl.ds(base, CHUNK)], x_v)
    pltpu.sync_copy(b_hbm.at[pl.ds(base, CHUNK)], b_v)

    zeros16 = jnp.zeros((LANES,), jnp.float32)

    @plsc.parallel_loop(0, N_GRAPHS, unroll=4)
    def _(i):
        acc_v[pl.ds(i * LANES, LANES)] = zeros16

    iota = lax.iota(jnp.int32, LANES)
    dnums = lax.GatherDimensionNumbers(
        offset_dims=(), collapsed_slice_dims=(0,), start_index_map=(0,))

    def _splat(vec, nn):
        return lax.gather(
            vec, jnp.full((LANES, 1), nn, jnp.int32), dnums, (1,),
            mode=lax.GatherScatterMode.PROMISE_IN_BOUNDS)

    @plsc.parallel_loop(0, CHUNK // LANES)
    def _(g):
        off = g * LANES
        xv16 = x_v[pl.ds(off, LANES)] * N_GEN
        bv16 = b_v[pl.ds(off, LANES)] * N_GEN
        for nn in range(LANES):
            row = plsc.load_gather(table_v, [_splat(xv16, nn) + iota])
            plsc.addupdate_scatter(
                acc_v, [_splat(bv16, nn) + iota], row)

    plsc.subcore_barrier()
    pltpu.sync_copy(acc_v, shared.at[sid])
    plsc.subcore_barrier()

    span = N_GRAPHS * N_GEN // NS
    for s in range(NS):
        pltpu.sync_copy(shared.at[s, pl.ds(sid * span, span)], buf_v)
        for r in range(span // LANES):
            sl = pl.ds(r * LANES, LANES)
            if s == 0:
                sum_v[sl] = buf_v[sl]
            else:
                sum_v[sl] = sum_v[sl] + buf_v[sl]
    pltpu.sync_copy(sum_v, out_hbm.at[pl.ds(sid * span, span)])


def _make_sc():
    mesh = plsc.VectorSubcoreMesh(
        core_axis_name="c", subcore_axis_name="s", num_cores=1)
    span = N_GRAPHS * N_GEN // NS
    return pl.kernel(
        _sc_body,
        out_type=jax.ShapeDtypeStruct((N_GRAPHS * N_GEN,), jnp.float32),
        mesh=mesh,
        scratch_types=[
            pltpu.VMEM((TABLE_ROWS * N_GEN,), jnp.float32),
            pltpu.VMEM((CHUNK,), jnp.int32),
            pltpu.VMEM((CHUNK,), jnp.int32),
            pltpu.VMEM((N_GRAPHS * N_GEN,), jnp.float32),
            pltpu.VMEM((span,), jnp.float32),
            pltpu.VMEM((span,), jnp.float32),
            pltpu.VMEM_SHARED((NS, N_GRAPHS * N_GEN), jnp.float32),
        ],
        compiler_params=pltpu.CompilerParams(
            use_tc_tiling_on_sc=False, needs_layout_passes=False),
    )


@jax.jit
def kernel(x, edge_index, batch, B, Pi):
    del edge_index
    n = x.shape[0]
    npad = NS * CHUNK
    pad = npad - n
    x_pad = jnp.concatenate([x, jnp.full((pad,), M, jnp.int32)])
    b_pad = jnp.concatenate([batch, jnp.full((pad,), batch[-1], jnp.int32)])

    table = _make_table(jnp.transpose(B, (0, 2, 1)), Pi)
    out = _make_sc()(table.reshape(-1), x_pad, b_pad)
    return out.reshape(N_GRAPHS, 1, N_GEN)
